# baseline (speedup 1.0000x reference)
import jax
import jax.numpy as jnp
import numpy as np
from jax import lax
from jax.experimental import pallas as pl
from jax.experimental.pallas import tpu as pltpu

_TUNNEL_PROB = 1.0 / 137.035999139


def _np_threefry_uniform(seed, n):
    m = np.uint64(0xFFFFFFFF)

    def rotl(x, d):
        return ((x << np.uint64(d)) | (x >> np.uint64(32 - d))) & m

    k0 = np.uint64(np.uint64(seed) >> np.uint64(32))
    k1 = np.uint64(np.uint64(seed) & m)
    ks2 = k0 ^ k1 ^ np.uint64(0x1BD11BDA)
    c64 = np.arange(n, dtype=np.uint64)
    x0 = (c64 >> np.uint64(32)) + k0 & m
    x1 = (c64 & m) + k1 & m
    keys = [(k1, ks2), (ks2, k0), (k0, k1), (k1, ks2), (ks2, k0)]
    rots = ([13, 15, 26, 6], [17, 29, 16, 24])
    for i in range(5):
        for r in rots[i % 2]:
            x0 = (x0 + x1) & m
            x1 = rotl(x1, r) ^ x0
        ka, kb = keys[i]
        x0 = (x0 + ka) & m
        x1 = (x1 + kb + np.uint64(i + 1)) & m
    bits = (x0 ^ x1).astype(np.uint32)
    fb = (bits >> np.uint32(9)) | np.uint32(0x3F800000)
    return fb.view(np.float32) - np.float32(1.0)


_TUNNEL_MASK = _np_threefry_uniform(1234, 4) < _TUNNEL_PROB

_BS = 512


def _router_body(x0_ref, x1_ref, c_ref, scores_ref, idx_ref, acc_ref, sums_ref):
    b = pl.program_id(0)
    j = pl.program_id(1)
    nb = pl.num_programs(0)
    ns = pl.num_programs(1)

    @pl.when(j == 0)
    def _():
        acc_ref[...] = jnp.zeros_like(acc_ref)

    h = x0_ref.shape[2]
    acc_ref[:, pl.ds(0, h)] += jnp.sum(x0_ref[0], axis=0, keepdims=True)
    acc_ref[:, pl.ds(h, h)] += jnp.sum(x1_ref[0], axis=0, keepdims=True)

    @pl.when(j == ns - 1)
    def _():
        sums_ref[pl.ds(b, 1), :] = acc_ref[...]

    @pl.when((b == nb - 1) & (j == ns - 1))
    def _():
        seq = x0_ref.shape[1] * ns
        e = c_ref.shape[0]
        q = sums_ref[...] * (1.0 / seq)
        qn = q / jnp.maximum(
            jnp.sqrt(jnp.sum(q * q, axis=-1, keepdims=True)), 1e-12)
        c = c_ref[...]
        cn = c / jnp.maximum(
            jnp.sqrt(jnp.sum(c * c, axis=-1, keepdims=True)), 1e-12)
        scores = lax.dot_general(
            qn, cn, (((1,), (1,)), ((), ())),
            preferred_element_type=jnp.float32)
        idx = lax.broadcasted_iota(jnp.int32, (nb, e), 1)
        max1 = jnp.max(scores, axis=1, keepdims=True)
        i1 = jnp.min(jnp.where(scores == max1, idx, e), axis=1, keepdims=True)
        masked = jnp.where(idx == i1, -jnp.inf, scores)
        max2 = jnp.max(masked, axis=1, keepdims=True)
        i2 = jnp.min(jnp.where(masked == max2, idx, e), axis=1, keepdims=True)
        minv = jnp.min(scores, axis=1, keepdims=True)
        imin = jnp.min(jnp.where(scores == minv, idx, e),
                       axis=1, keepdims=True)
        rows_i = lax.broadcasted_iota(jnp.int32, (nb, 1), 0)
        tunnel = jnp.zeros((nb, 1), jnp.bool_)
        for k, msk in enumerate(_TUNNEL_MASK.tolist()):
            if msk:
                tunnel = jnp.logical_or(tunnel, rows_i == k)
        top0 = jnp.where(tunnel, imin, i1)
        scores_ref[...] = jnp.where((idx == 0) & tunnel, minv, scores)
        idx_ref[...] = jnp.concatenate([top0, i2], axis=1).astype(jnp.int32)


def kernel(x, centroids):
    bsz, seq, d = x.shape
    e = centroids.shape[0]
    ns = seq // _BS
    scores_t, top_idx = pl.pallas_call(
        _router_body,
        grid=(bsz, ns),
        in_specs=[
            pl.BlockSpec((1, _BS, d // 2), lambda b, j: (b, j, 0)),
            pl.BlockSpec((1, _BS, d // 2), lambda b, j: (b, j, 1)),
            pl.BlockSpec((e, d), lambda b, j: (0, 0)),
        ],
        out_specs=[
            pl.BlockSpec((bsz, e), lambda b, j: (0, 0)),
            pl.BlockSpec((bsz, 2), lambda b, j: (0, 0)),
        ],
        out_shape=[
            jax.ShapeDtypeStruct((bsz, e), jnp.float32),
            jax.ShapeDtypeStruct((bsz, 2), jnp.int32),
        ],
        scratch_shapes=[
            pltpu.VMEM((1, d), jnp.float32),
            pltpu.VMEM((bsz, d), jnp.float32),
        ],
        compiler_params=pltpu.CompilerParams(
            dimension_semantics=("arbitrary", "arbitrary")),
    )(x, x, centroids)
    return (scores_t, top_idx)


# speedup vs baseline: 1.0567x; 1.0567x over previous
import jax
import jax.numpy as jnp
import numpy as np
from jax import lax
from jax.experimental import pallas as pl
from jax.experimental.pallas import tpu as pltpu

_TUNNEL_PROB = 1.0 / 137.035999139


def _np_threefry_uniform(seed, n):
    m = np.uint64(0xFFFFFFFF)

    def rotl(x, d):
        return ((x << np.uint64(d)) | (x >> np.uint64(32 - d))) & m

    k0 = np.uint64(np.uint64(seed) >> np.uint64(32))
    k1 = np.uint64(np.uint64(seed) & m)
    ks2 = k0 ^ k1 ^ np.uint64(0x1BD11BDA)
    c64 = np.arange(n, dtype=np.uint64)
    x0 = (c64 >> np.uint64(32)) + k0 & m
    x1 = (c64 & m) + k1 & m
    keys = [(k1, ks2), (ks2, k0), (k0, k1), (k1, ks2), (ks2, k0)]
    rots = ([13, 15, 26, 6], [17, 29, 16, 24])
    for i in range(5):
        for r in rots[i % 2]:
            x0 = (x0 + x1) & m
            x1 = rotl(x1, r) ^ x0
        ka, kb = keys[i]
        x0 = (x0 + ka) & m
        x1 = (x1 + kb + np.uint64(i + 1)) & m
    bits = (x0 ^ x1).astype(np.uint32)
    fb = (bits >> np.uint32(9)) | np.uint32(0x3F800000)
    return fb.view(np.float32) - np.float32(1.0)


_TUNNEL_MASK = _np_threefry_uniform(1234, 4) < _TUNNEL_PROB

_BS = 1024


def _router_body(x_ref, c_ref, scores_ref, idx_ref, acc_ref, sums_ref):
    b = pl.program_id(0)
    j = pl.program_id(1)
    nb = pl.num_programs(0)
    ns = pl.num_programs(1)

    @pl.when(j == 0)
    def _():
        acc_ref[...] = jnp.zeros_like(acc_ref)

    acc_ref[...] += jnp.sum(x_ref[0], axis=0, keepdims=True)

    @pl.when(j == ns - 1)
    def _():
        sums_ref[pl.ds(b, 1), :] = acc_ref[...]

    @pl.when((b == nb - 1) & (j == ns - 1))
    def _():
        seq = x_ref.shape[1] * ns
        e = c_ref.shape[0]
        q = sums_ref[...] * (1.0 / seq)
        qn = q / jnp.maximum(
            jnp.sqrt(jnp.sum(q * q, axis=-1, keepdims=True)), 1e-12)
        c = c_ref[...]
        cn = c / jnp.maximum(
            jnp.sqrt(jnp.sum(c * c, axis=-1, keepdims=True)), 1e-12)
        scores = lax.dot_general(
            qn, cn, (((1,), (1,)), ((), ())),
            preferred_element_type=jnp.float32)
        idx = lax.broadcasted_iota(jnp.int32, (nb, e), 1)
        max1 = jnp.max(scores, axis=1, keepdims=True)
        i1 = jnp.min(jnp.where(scores == max1, idx, e), axis=1, keepdims=True)
        masked = jnp.where(idx == i1, -jnp.inf, scores)
        max2 = jnp.max(masked, axis=1, keepdims=True)
        i2 = jnp.min(jnp.where(masked == max2, idx, e), axis=1, keepdims=True)
        minv = jnp.min(scores, axis=1, keepdims=True)
        imin = jnp.min(jnp.where(scores == minv, idx, e),
                       axis=1, keepdims=True)
        rows_i = lax.broadcasted_iota(jnp.int32, (nb, 1), 0)
        tunnel = jnp.zeros((nb, 1), jnp.bool_)
        for k, msk in enumerate(_TUNNEL_MASK.tolist()):
            if msk:
                tunnel = jnp.logical_or(tunnel, rows_i == k)
        top0 = jnp.where(tunnel, imin, i1)
        scores_ref[...] = jnp.where((idx == 0) & tunnel, minv, scores)
        idx_ref[...] = jnp.concatenate([top0, i2], axis=1).astype(jnp.int32)


def kernel(x, centroids):
    bsz, seq, d = x.shape
    e = centroids.shape[0]
    ns = seq // _BS
    scores_t, top_idx = pl.pallas_call(
        _router_body,
        grid=(bsz, ns),
        in_specs=[
            pl.BlockSpec((1, _BS, d), lambda b, j: (b, j, 0)),
            pl.BlockSpec((e, d), lambda b, j: (0, 0)),
        ],
        out_specs=[
            pl.BlockSpec((bsz, e), lambda b, j: (0, 0)),
            pl.BlockSpec((bsz, 2), lambda b, j: (0, 0)),
        ],
        out_shape=[
            jax.ShapeDtypeStruct((bsz, e), jnp.float32),
            jax.ShapeDtypeStruct((bsz, 2), jnp.int32),
        ],
        scratch_shapes=[
            pltpu.VMEM((1, d), jnp.float32),
            pltpu.VMEM((bsz, d), jnp.float32),
        ],
        compiler_params=pltpu.CompilerParams(
            dimension_semantics=("arbitrary", "arbitrary")),
    )(x, centroids)
    return (scores_t, top_idx)
